# baseline (speedup 1.0000x reference)
import jax
import jax.numpy as jnp
from jax.experimental import pallas as pl
from jax.experimental.pallas import tpu as pltpu

B = 16384
D_IN = 128
D_OUT = 128
CHUNKS = (4096, 4096, 4096, 4096)
assert sum(CHUNKS) == B
_OFFS = tuple(sum(CHUNKS[:k]) for k in range(len(CHUNKS)))
_N = len(CHUNKS)


def _in_copy(data_hbm, dbufs, isem, k):
    return pltpu.make_async_copy(
        data_hbm.at[pl.ds(_OFFS[k], CHUNKS[k]), :], dbufs[k], isem.at[k]
    )


def _out_copy(o_hbm, obufs, osem, k):
    return pltpu.make_async_copy(
        obufs[k], o_hbm.at[pl.ds(_OFFS[k], CHUNKS[k]), :], osem.at[k]
    )


def _body(ind_hbm, data_hbm, w_hbm, b_hbm, o_hbm, *scratch):
    ind_v, w_v, b_v = scratch[0], scratch[1], scratch[2]
    dbufs = scratch[3:3 + _N]
    obufs = scratch[3 + _N:3 + 2 * _N]
    csem, isem, osem = scratch[3 + 2 * _N:]

    cp_ind = pltpu.make_async_copy(ind_hbm, ind_v, csem.at[0])
    cp_w = pltpu.make_async_copy(w_hbm, w_v, csem.at[1])
    cp_b = pltpu.make_async_copy(b_hbm, b_v, csem.at[2])
    for k in range(_N):
        _in_copy(data_hbm, dbufs, isem, k).start()
    cp_ind.start()
    cp_w.start()
    cp_b.start()
    cp_ind.wait()
    cp_w.wait()
    cp_b.wait()
    for k in range(_N):
        _in_copy(data_hbm, dbufs, isem, k).wait()
        q = CHUNKS[k] // 128
        y = jnp.dot(dbufs[k][...], w_v[...], preferred_element_type=jnp.float32)
        y = y + b_v[...]
        mf = (ind_v[pl.ds(_OFFS[k] // 128, q), :] != 0).astype(jnp.float32)
        y3 = y.reshape(q, 128, D_OUT)
        masked = y3 * mf[:, :, None]
        obufs[k][...] = masked.reshape(CHUNKS[k], D_OUT)
        _out_copy(o_hbm, obufs, osem, k).start()
    for k in range(_N):
        _out_copy(o_hbm, obufs, osem, k).wait()


def kernel(indicators, data, W, b):
    ind2d = indicators.reshape(128, 128)
    b2d = b.reshape(1, D_OUT)
    hbm = pl.BlockSpec(memory_space=pltpu.MemorySpace.HBM)
    scratch = [
        pltpu.VMEM((128, 128), jnp.int32),
        pltpu.VMEM((D_IN, D_OUT), jnp.float32),
        pltpu.VMEM((1, D_OUT), jnp.float32),
    ]
    scratch += [pltpu.VMEM((c, D_IN), jnp.float32) for c in CHUNKS]
    scratch += [pltpu.VMEM((c, D_OUT), jnp.float32) for c in CHUNKS]
    scratch += [
        pltpu.SemaphoreType.DMA((3,)),
        pltpu.SemaphoreType.DMA((_N,)),
        pltpu.SemaphoreType.DMA((_N,)),
    ]
    return pl.pallas_call(
        _body,
        in_specs=[hbm, hbm, hbm, hbm],
        out_specs=hbm,
        out_shape=jax.ShapeDtypeStruct((B, D_OUT), jnp.float32),
        scratch_shapes=scratch,
    )(ind2d, data, W, b2d)


# speedup vs baseline: 1.1070x; 1.1070x over previous
import jax
import jax.numpy as jnp
from jax.experimental import pallas as pl
from jax.experimental.pallas import tpu as pltpu

B = 16384
D_IN = 128
D_OUT = 128
CHUNKS = (4096, 4096, 4096, 4096)
assert sum(CHUNKS) == B
_OFFS = tuple(sum(CHUNKS[:k]) for k in range(len(CHUNKS)))
_N = len(CHUNKS)


def _in_copy(data_hbm, dbufs, isem, k):
    return pltpu.make_async_copy(
        data_hbm.at[pl.ds(_OFFS[k], CHUNKS[k]), :], dbufs[k], isem.at[k]
    )


def _out_copy(o_hbm, obufs, osem, k):
    return pltpu.make_async_copy(
        obufs[k], o_hbm.at[pl.ds(_OFFS[k], CHUNKS[k]), :], osem.at[k]
    )


def _body(ind_hbm, data_hbm, w_hbm, b_hbm, o_hbm, *scratch):
    ind_v, w_v, b_v = scratch[0], scratch[1], scratch[2]
    dbufs = scratch[3:3 + _N]
    obufs = scratch[3 + _N:3 + 2 * _N]
    csem, isem, osem = scratch[3 + 2 * _N:]

    cp_ind = pltpu.make_async_copy(ind_hbm, ind_v, csem.at[0])
    cp_w = pltpu.make_async_copy(w_hbm, w_v, csem.at[1])
    cp_b = pltpu.make_async_copy(b_hbm, b_v, csem.at[2])
    cp_ind.start()
    cp_w.start()
    cp_b.start()
    for k in range(_N):
        _in_copy(data_hbm, dbufs, isem, k).start()
    cp_ind.wait()
    cp_w.wait()
    cp_b.wait()
    for k in range(_N):
        _in_copy(data_hbm, dbufs, isem, k).wait()
        halves = 2 if k == _N - 1 else 1
        part = CHUNKS[k] // halves
        for h in range(halves):
            q = part // 128
            lo = h * part
            x = dbufs[k][pl.ds(lo, part), :]
            y = jnp.dot(x, w_v[...], preferred_element_type=jnp.float32)
            y = y + b_v[...]
            mf = (ind_v[pl.ds((_OFFS[k] + lo) // 128, q), :] != 0)
            mf = mf.astype(jnp.float32)
            y3 = y.reshape(q, 128, D_OUT)
            masked = y3 * mf[:, :, None]
            obufs[k][pl.ds(lo, part), :] = masked.reshape(part, D_OUT)
            pltpu.make_async_copy(
                obufs[k].at[pl.ds(lo, part), :],
                o_hbm.at[pl.ds(_OFFS[k] + lo, part), :],
                osem.at[k],
            ).start()
    for k in range(_N):
        _out_copy(o_hbm, obufs, osem, k).wait()


def kernel(indicators, data, W, b):
    ind2d = indicators.reshape(128, 128)
    b2d = b.reshape(1, D_OUT)
    hbm = pl.BlockSpec(memory_space=pltpu.MemorySpace.HBM)
    scratch = [
        pltpu.VMEM((128, 128), jnp.int32),
        pltpu.VMEM((D_IN, D_OUT), jnp.float32),
        pltpu.VMEM((1, D_OUT), jnp.float32),
    ]
    scratch += [pltpu.VMEM((c, D_IN), jnp.float32) for c in CHUNKS]
    scratch += [pltpu.VMEM((c, D_OUT), jnp.float32) for c in CHUNKS]
    scratch += [
        pltpu.SemaphoreType.DMA((3,)),
        pltpu.SemaphoreType.DMA((_N,)),
        pltpu.SemaphoreType.DMA((_N,)),
    ]
    return pl.pallas_call(
        _body,
        in_specs=[hbm, hbm, hbm, hbm],
        out_specs=hbm,
        out_shape=jax.ShapeDtypeStruct((B, D_OUT), jnp.float32),
        scratch_shapes=scratch,
    )(ind2d, data, W, b2d)


# speedup vs baseline: 1.1086x; 1.0015x over previous
import jax
import jax.numpy as jnp
from jax.experimental import pallas as pl
from jax.experimental.pallas import tpu as pltpu

B = 16384
D_IN = 128
D_OUT = 128
CHUNKS = (4096, 4096, 4096, 4096)
assert sum(CHUNKS) == B
_OFFS = tuple(sum(CHUNKS[:k]) for k in range(len(CHUNKS)))
_N = len(CHUNKS)


def _in_copy(data_hbm, dbufs, isem, k):
    return pltpu.make_async_copy(
        data_hbm.at[pl.ds(_OFFS[k], CHUNKS[k]), :], dbufs[k], isem.at[k]
    )


def _out_copy(o_hbm, obufs, osem, k):
    return pltpu.make_async_copy(
        obufs[k], o_hbm.at[pl.ds(_OFFS[k], CHUNKS[k]), :], osem.at[k]
    )


def _body(ind_hbm, data_hbm, w_hbm, b_hbm, o_hbm, *scratch):
    ind_v, w_v, b_v = scratch[0], scratch[1], scratch[2]
    dbufs = scratch[3:3 + _N]
    obufs = scratch[3 + _N:3 + 2 * _N]
    csem, isem, osem = scratch[3 + 2 * _N:]

    cp_ind = pltpu.make_async_copy(ind_hbm, ind_v, csem.at[0])
    cp_w = pltpu.make_async_copy(w_hbm, w_v, csem.at[1])
    cp_b = pltpu.make_async_copy(b_hbm, b_v, csem.at[2])
    cp_ind.start()
    cp_w.start()
    cp_b.start()
    for k in range(_N):
        _in_copy(data_hbm, dbufs, isem, k).start()
    cp_ind.wait()
    cp_w.wait()
    cp_b.wait()
    for k in range(_N):
        _in_copy(data_hbm, dbufs, isem, k).wait()
        halves = 2 if k >= _N - 2 else 1
        part = CHUNKS[k] // halves
        for h in range(halves):
            q = part // 128
            lo = h * part
            x = dbufs[k][pl.ds(lo, part), :]
            y = jnp.dot(x, w_v[...], preferred_element_type=jnp.float32)
            y = y + b_v[...]
            mf = (ind_v[pl.ds((_OFFS[k] + lo) // 128, q), :] != 0)
            mf = mf.astype(jnp.float32)
            y3 = y.reshape(q, 128, D_OUT)
            masked = y3 * mf[:, :, None]
            obufs[k][pl.ds(lo, part), :] = masked.reshape(part, D_OUT)
            pltpu.make_async_copy(
                obufs[k].at[pl.ds(lo, part), :],
                o_hbm.at[pl.ds(_OFFS[k] + lo, part), :],
                osem.at[k],
            ).start()
    for k in range(_N):
        _out_copy(o_hbm, obufs, osem, k).wait()


def kernel(indicators, data, W, b):
    ind2d = indicators.reshape(128, 128)
    b2d = b.reshape(1, D_OUT)
    hbm = pl.BlockSpec(memory_space=pltpu.MemorySpace.HBM)
    scratch = [
        pltpu.VMEM((128, 128), jnp.int32),
        pltpu.VMEM((D_IN, D_OUT), jnp.float32),
        pltpu.VMEM((1, D_OUT), jnp.float32),
    ]
    scratch += [pltpu.VMEM((c, D_IN), jnp.float32) for c in CHUNKS]
    scratch += [pltpu.VMEM((c, D_OUT), jnp.float32) for c in CHUNKS]
    scratch += [
        pltpu.SemaphoreType.DMA((3,)),
        pltpu.SemaphoreType.DMA((_N,)),
        pltpu.SemaphoreType.DMA((_N,)),
    ]
    return pl.pallas_call(
        _body,
        in_specs=[hbm, hbm, hbm, hbm],
        out_specs=hbm,
        out_shape=jax.ShapeDtypeStruct((B, D_OUT), jnp.float32),
        scratch_shapes=scratch,
    )(ind2d, data, W, b2d)


# speedup vs baseline: 1.1378x; 1.0263x over previous
import jax
import jax.numpy as jnp
from jax.experimental import pallas as pl
from jax.experimental.pallas import tpu as pltpu

B = 16384
D_IN = 128
D_OUT = 128
CHUNKS = (4096, 4096, 4096, 4096)
assert sum(CHUNKS) == B
_OFFS = tuple(sum(CHUNKS[:k]) for k in range(len(CHUNKS)))
_N = len(CHUNKS)


def _in_copy(data_hbm, dbufs, isem, k):
    return pltpu.make_async_copy(
        data_hbm.at[pl.ds(_OFFS[k], CHUNKS[k]), :], dbufs[k], isem.at[k]
    )


def _out_copy(o_hbm, obufs, osem, k):
    return pltpu.make_async_copy(
        obufs[k], o_hbm.at[pl.ds(_OFFS[k], CHUNKS[k]), :], osem.at[k]
    )


def _body(ind_hbm, data_hbm, w_hbm, b_hbm, o_hbm, *scratch):
    ind_v, w_v, b_v = scratch[0], scratch[1], scratch[2]
    dbufs = scratch[3:3 + _N]
    obufs = scratch[3 + _N:3 + 2 * _N]
    csem, isem, osem = scratch[3 + 2 * _N:]

    cp_ind = pltpu.make_async_copy(ind_hbm, ind_v, csem.at[0])
    cp_w = pltpu.make_async_copy(w_hbm, w_v, csem.at[1])
    cp_b = pltpu.make_async_copy(b_hbm, b_v, csem.at[2])
    cp_ind.start()
    cp_w.start()
    cp_b.start()
    for k in range(_N):
        _in_copy(data_hbm, dbufs, isem, k).start()
    cp_ind.wait()
    cp_w.wait()
    cp_b.wait()
    for k in range(_N):
        _in_copy(data_hbm, dbufs, isem, k).wait()
        halves = 2
        part = CHUNKS[k] // halves
        for h in range(halves):
            q = part // 128
            lo = h * part
            x = dbufs[k][pl.ds(lo, part), :]
            y = jnp.dot(x, w_v[...], preferred_element_type=jnp.float32)
            y = y + b_v[...]
            mf = (ind_v[pl.ds((_OFFS[k] + lo) // 128, q), :] != 0)
            mf = mf.astype(jnp.float32)
            y3 = y.reshape(q, 128, D_OUT)
            masked = y3 * mf[:, :, None]
            obufs[k][pl.ds(lo, part), :] = masked.reshape(part, D_OUT)
            pltpu.make_async_copy(
                obufs[k].at[pl.ds(lo, part), :],
                o_hbm.at[pl.ds(_OFFS[k] + lo, part), :],
                osem.at[k],
            ).start()
    for k in range(_N):
        _out_copy(o_hbm, obufs, osem, k).wait()


def kernel(indicators, data, W, b):
    ind2d = indicators.reshape(128, 128)
    b2d = b.reshape(1, D_OUT)
    hbm = pl.BlockSpec(memory_space=pltpu.MemorySpace.HBM)
    scratch = [
        pltpu.VMEM((128, 128), jnp.int32),
        pltpu.VMEM((D_IN, D_OUT), jnp.float32),
        pltpu.VMEM((1, D_OUT), jnp.float32),
    ]
    scratch += [pltpu.VMEM((c, D_IN), jnp.float32) for c in CHUNKS]
    scratch += [pltpu.VMEM((c, D_OUT), jnp.float32) for c in CHUNKS]
    scratch += [
        pltpu.SemaphoreType.DMA((3,)),
        pltpu.SemaphoreType.DMA((_N,)),
        pltpu.SemaphoreType.DMA((_N,)),
    ]
    return pl.pallas_call(
        _body,
        in_specs=[hbm, hbm, hbm, hbm],
        out_specs=hbm,
        out_shape=jax.ShapeDtypeStruct((B, D_OUT), jnp.float32),
        scratch_shapes=scratch,
    )(ind2d, data, W, b2d)


# speedup vs baseline: 1.1399x; 1.0019x over previous
import jax
import jax.numpy as jnp
from jax.experimental import pallas as pl
from jax.experimental.pallas import tpu as pltpu

B = 16384
D_IN = 128
D_OUT = 128
CHUNKS = (4096, 4096, 4096, 4096)
assert sum(CHUNKS) == B
_OFFS = tuple(sum(CHUNKS[:k]) for k in range(len(CHUNKS)))
_N = len(CHUNKS)


def _in_copy(data_hbm, dbufs, isem, k):
    return pltpu.make_async_copy(
        data_hbm.at[pl.ds(_OFFS[k], CHUNKS[k]), :], dbufs[k], isem.at[k]
    )


def _out_copy(o_hbm, obufs, osem, k):
    return pltpu.make_async_copy(
        obufs[k], o_hbm.at[pl.ds(_OFFS[k], CHUNKS[k]), :], osem.at[k]
    )


def _body(ind_hbm, data_hbm, w_hbm, b_hbm, o_hbm, *scratch):
    ind_v, w_v, b_v = scratch[0], scratch[1], scratch[2]
    dbufs = scratch[3:3 + _N]
    obufs = scratch[3 + _N:3 + 2 * _N]
    csem, isem, osem = scratch[3 + 2 * _N:]

    cp_ind = pltpu.make_async_copy(ind_hbm, ind_v, csem.at[0])
    cp_w = pltpu.make_async_copy(w_hbm, w_v, csem.at[1])
    cp_b = pltpu.make_async_copy(b_hbm, b_v, csem.at[2])
    cp_ind.start()
    cp_w.start()
    cp_b.start()
    for k in range(_N):
        _in_copy(data_hbm, dbufs, isem, k).start()
    cp_ind.wait()
    cp_w.wait()
    cp_b.wait()
    for k in range(_N):
        _in_copy(data_hbm, dbufs, isem, k).wait()
        halves = 4
        part = CHUNKS[k] // halves
        for h in range(halves):
            q = part // 128
            lo = h * part
            x = dbufs[k][pl.ds(lo, part), :]
            y = jnp.dot(x, w_v[...], preferred_element_type=jnp.float32)
            y = y + b_v[...]
            mf = (ind_v[pl.ds((_OFFS[k] + lo) // 128, q), :] != 0)
            mf = mf.astype(jnp.float32)
            y3 = y.reshape(q, 128, D_OUT)
            masked = y3 * mf[:, :, None]
            obufs[k][pl.ds(lo, part), :] = masked.reshape(part, D_OUT)
            pltpu.make_async_copy(
                obufs[k].at[pl.ds(lo, part), :],
                o_hbm.at[pl.ds(_OFFS[k] + lo, part), :],
                osem.at[k],
            ).start()
    for k in range(_N):
        _out_copy(o_hbm, obufs, osem, k).wait()


def kernel(indicators, data, W, b):
    ind2d = indicators.reshape(128, 128)
    b2d = b.reshape(1, D_OUT)
    hbm = pl.BlockSpec(memory_space=pltpu.MemorySpace.HBM)
    scratch = [
        pltpu.VMEM((128, 128), jnp.int32),
        pltpu.VMEM((D_IN, D_OUT), jnp.float32),
        pltpu.VMEM((1, D_OUT), jnp.float32),
    ]
    scratch += [pltpu.VMEM((c, D_IN), jnp.float32) for c in CHUNKS]
    scratch += [pltpu.VMEM((c, D_OUT), jnp.float32) for c in CHUNKS]
    scratch += [
        pltpu.SemaphoreType.DMA((3,)),
        pltpu.SemaphoreType.DMA((_N,)),
        pltpu.SemaphoreType.DMA((_N,)),
    ]
    return pl.pallas_call(
        _body,
        in_specs=[hbm, hbm, hbm, hbm],
        out_specs=hbm,
        out_shape=jax.ShapeDtypeStruct((B, D_OUT), jnp.float32),
        scratch_shapes=scratch,
    )(ind2d, data, W, b2d)


# speedup vs baseline: 1.1808x; 1.0359x over previous
import jax
import jax.numpy as jnp
from jax.experimental import pallas as pl
from jax.experimental.pallas import tpu as pltpu

B = 16384
D_IN = 128
D_OUT = 128
CHUNKS = (8192, 8192)
assert sum(CHUNKS) == B
_OFFS = tuple(sum(CHUNKS[:k]) for k in range(len(CHUNKS)))
_N = len(CHUNKS)


def _in_copy(data_hbm, dbufs, isem, k):
    return pltpu.make_async_copy(
        data_hbm.at[pl.ds(_OFFS[k], CHUNKS[k]), :], dbufs[k], isem.at[k]
    )


def _out_copy(o_hbm, obufs, osem, k):
    return pltpu.make_async_copy(
        obufs[k], o_hbm.at[pl.ds(_OFFS[k], CHUNKS[k]), :], osem.at[k]
    )


def _body(ind_hbm, data_hbm, w_hbm, b_hbm, o_hbm, *scratch):
    ind_v, w_v, b_v = scratch[0], scratch[1], scratch[2]
    dbufs = scratch[3:3 + _N]
    obufs = scratch[3 + _N:3 + 2 * _N]
    csem, isem, osem = scratch[3 + 2 * _N:]

    cp_ind = pltpu.make_async_copy(ind_hbm, ind_v, csem.at[0])
    cp_w = pltpu.make_async_copy(w_hbm, w_v, csem.at[1])
    cp_b = pltpu.make_async_copy(b_hbm, b_v, csem.at[2])
    cp_ind.start()
    cp_w.start()
    cp_b.start()
    for k in range(_N):
        _in_copy(data_hbm, dbufs, isem, k).start()
    cp_ind.wait()
    cp_w.wait()
    cp_b.wait()
    for k in range(_N):
        _in_copy(data_hbm, dbufs, isem, k).wait()
        halves = 4
        part = CHUNKS[k] // halves
        for h in range(halves):
            q = part // 128
            lo = h * part
            x = dbufs[k][pl.ds(lo, part), :]
            y = jnp.dot(x, w_v[...], preferred_element_type=jnp.float32)
            y = y + b_v[...]
            mf = (ind_v[pl.ds((_OFFS[k] + lo) // 128, q), :] != 0)
            mf = mf.astype(jnp.float32)
            y3 = y.reshape(q, 128, D_OUT)
            masked = y3 * mf[:, :, None]
            obufs[k][pl.ds(lo, part), :] = masked.reshape(part, D_OUT)
            pltpu.make_async_copy(
                obufs[k].at[pl.ds(lo, part), :],
                o_hbm.at[pl.ds(_OFFS[k] + lo, part), :],
                osem.at[k],
            ).start()
    for k in range(_N):
        _out_copy(o_hbm, obufs, osem, k).wait()


def kernel(indicators, data, W, b):
    ind2d = indicators.reshape(128, 128)
    b2d = b.reshape(1, D_OUT)
    hbm = pl.BlockSpec(memory_space=pltpu.MemorySpace.HBM)
    scratch = [
        pltpu.VMEM((128, 128), jnp.int32),
        pltpu.VMEM((D_IN, D_OUT), jnp.float32),
        pltpu.VMEM((1, D_OUT), jnp.float32),
    ]
    scratch += [pltpu.VMEM((c, D_IN), jnp.float32) for c in CHUNKS]
    scratch += [pltpu.VMEM((c, D_OUT), jnp.float32) for c in CHUNKS]
    scratch += [
        pltpu.SemaphoreType.DMA((3,)),
        pltpu.SemaphoreType.DMA((_N,)),
        pltpu.SemaphoreType.DMA((_N,)),
    ]
    return pl.pallas_call(
        _body,
        in_specs=[hbm, hbm, hbm, hbm],
        out_specs=hbm,
        out_shape=jax.ShapeDtypeStruct((B, D_OUT), jnp.float32),
        scratch_shapes=scratch,
    )(ind2d, data, W, b2d)
